# baseline (speedup 1.0000x reference)
import functools

import jax
import jax.numpy as jnp
import numpy as np
from jax import lax
from jax.experimental import pallas as pl
from jax.experimental.pallas import tpu as pltpu
from jax.experimental.pallas import tpu_sc as plsc


def _cut_box(H, W, alpha=1.0, seed=0):
    rng = np.random.RandomState(seed)
    lam = rng.beta(alpha, alpha)
    cx = rng.uniform(0, W)
    cy = rng.uniform(0, H)
    w = W * np.sqrt(1.0 - lam)
    h = H * np.sqrt(1.0 - lam)
    x0 = int(np.clip(cx - w // 2, 0, W))
    y0 = int(np.clip(cy - h // 2, 0, H))
    x1 = int(np.clip(cx + w // 2, 0, W))
    y1 = int(np.clip(cy + h // 2, 0, H))
    return x0, y0, x1, y1


_Y0 = 103
_X1 = 87
_RS = 96


def _sc_body(B, C, H, W, NW, img_ref, index_ref, labels_ref,
             out_ref, lab_out_ref,
             bufA, bufB, bufP, idxbuf, indexv, labelsv, laboutv, sem):
    nc = plsc.get_sparse_core_info().num_cores
    wid = lax.axis_index("s") * nc + lax.axis_index("c")
    planes_per_w = (B * C) // NW
    NB = H - _RS
    NA = _RS

    pltpu.sync_copy(index_ref, indexv)

    iota = lax.iota(jnp.int32, 16)

    def do_plane(i, _):
        p = wid * planes_per_w + i
        b = p // C
        c = p - b * C
        row0 = p * H

        pltpu.sync_copy(img_ref.at[pl.ds(row0, NA)], bufA)
        pltpu.sync_copy(bufA, out_ref.at[pl.ds(row0, NA)])

        pltpu.sync_copy(img_ref.at[pl.ds(row0 + _RS, NB)], bufB)

        bvec = jnp.full((16,), b, dtype=jnp.int32)
        srcb = plsc.load_gather(indexv, [bvec])
        rowbase = srcb * (C * H) + c * H + _RS
        for k in range(NB // 16):
            idxbuf[pl.ds(k * 16, 16)] = rowbase + k * 16 + iota
        pltpu.async_copy(img_ref.at[idxbuf], bufP, sem).wait()

        j0 = _Y0 - _RS
        nfull = _X1 // 16
        ntail = _X1 - nfull * 16

        def blend(j, _):
            for k in range(nfull):
                bufB[j, pl.ds(k * 16, 16)] = bufP[j, pl.ds(k * 16, 16)]
            vp = bufP[j, pl.ds(nfull * 16, 16)]
            vs = bufB[j, pl.ds(nfull * 16, 16)]
            bufB[j, pl.ds(nfull * 16, 16)] = jnp.where(iota < ntail, vp, vs)
            return _
        lax.fori_loop(j0, NB, blend, 0)

        pltpu.sync_copy(bufB, out_ref.at[pl.ds(row0 + _RS, NB)])
        return _

    lax.fori_loop(0, planes_per_w, do_plane, 0)

    @pl.when(wid == 0)
    def _labels():
        pltpu.sync_copy(labels_ref, labelsv)
        for k in range(B // 16):
            idxv = indexv[pl.ds(k * 16, 16)]
            laboutv[pl.ds(k * 16, 16)] = plsc.load_gather(labelsv, [idxv])
        pltpu.sync_copy(laboutv, lab_out_ref)


def kernel(images, labels, index):
    B, C, H, W = images.shape
    x0, y0, x1, y1 = _cut_box(H, W, alpha=1.0, seed=0)
    assert (x0, y0, x1, y1) == (0, _Y0, _X1, H)

    info = plsc.get_sparse_core_info()
    NW = info.num_cores * info.num_subcores

    img2d = images.reshape(B * C * H, W)
    mesh = plsc.VectorSubcoreMesh(core_axis_name="c", subcore_axis_name="s")

    sc = pl.kernel(
        functools.partial(_sc_body, B, C, H, W, NW),
        out_type=[
            jax.ShapeDtypeStruct((B * C * H, W), images.dtype),
            jax.ShapeDtypeStruct((B,), labels.dtype),
        ],
        mesh=mesh,
        scratch_types=[
            pltpu.VMEM((_RS, W), jnp.float32),
            pltpu.VMEM((H - _RS, W), jnp.float32),
            pltpu.VMEM((H - _RS, W), jnp.float32),
            pltpu.VMEM((H - _RS,), jnp.int32),
            pltpu.VMEM((B,), jnp.int32),
            pltpu.VMEM((B,), jnp.int32),
            pltpu.VMEM((B,), jnp.int32),
            pltpu.SemaphoreType.DMA,
        ],
        compiler_params=pltpu.CompilerParams(
            needs_layout_passes=False, use_tc_tiling_on_sc=False),
    )
    out2d, labels_b = sc(img2d, index, labels)
    mixed = out2d.reshape(B, C, H, W)

    lam = 1.0 - (x1 - x0) * (y1 - y0) / (W * H)
    return (mixed, labels, labels_b, jnp.float32(lam))


# speedup vs baseline: 1.0735x; 1.0735x over previous
import functools

import jax
import jax.numpy as jnp
import numpy as np
from jax import lax
from jax.experimental import pallas as pl
from jax.experimental.pallas import tpu as pltpu
from jax.experimental.pallas import tpu_sc as plsc


def _cut_box(H, W, alpha=1.0, seed=0):
    rng = np.random.RandomState(seed)
    lam = rng.beta(alpha, alpha)
    cx = rng.uniform(0, W)
    cy = rng.uniform(0, H)
    w = W * np.sqrt(1.0 - lam)
    h = H * np.sqrt(1.0 - lam)
    x0 = int(np.clip(cx - w // 2, 0, W))
    y0 = int(np.clip(cy - h // 2, 0, H))
    x1 = int(np.clip(cx + w // 2, 0, W))
    y1 = int(np.clip(cy + h // 2, 0, H))
    return x0, y0, x1, y1


_Y0 = 103
_X1 = 87
_RS = 96
_NS = 4


def _sc_body(B, C, H, W, TPW, img_ref, index_ref, labels_ref,
             out_ref, lab_out_ref, *scratch):
    bufIn = scratch[0:_NS]
    bufP = scratch[_NS:2 * _NS]
    idxbuf = scratch[2 * _NS:3 * _NS]
    indexv, labelsv, laboutv = scratch[3 * _NS:3 * _NS + 3]
    semIn = scratch[3 * _NS + 3:4 * _NS + 3]
    semP = scratch[4 * _NS + 3:5 * _NS + 3]
    semOut = scratch[5 * _NS + 3:6 * _NS + 3]

    nc = plsc.get_sparse_core_info().num_cores
    wid = lax.axis_index("s") * nc + lax.axis_index("c")
    planes_per_w = TPW // 4
    iota = lax.iota(jnp.int32, 16)

    pltpu.sync_copy(index_ref, indexv)

    def tinfo(t):
        kind = t % 4
        p = wid * planes_per_w + t // 4
        rowoff = jnp.where(kind < 2, 48 * kind, 64 * kind - 32)
        return kind, p, p * H + rowoff

    def start_in(t, s):
        kind, p, g0 = tinfo(t)

        @pl.when(kind < 2)
        def _a():
            pltpu.make_async_copy(img_ref.at[pl.ds(g0, 48)],
                                  bufIn[s].at[pl.ds(0, 48)], semIn[s]).start()

        @pl.when(kind >= 2)
        def _b():
            pltpu.make_async_copy(img_ref.at[pl.ds(g0, 64)],
                                  bufIn[s].at[pl.ds(0, 64)], semIn[s]).start()
            b = p // C
            c = p - b * C
            srcb = plsc.load_gather(indexv, [jnp.full((16,), b, jnp.int32)])
            rowbase = srcb * (C * H) + c * H + (64 * kind - 32)
            for k in range(4):
                idxbuf[s][pl.ds(k * 16, 16)] = rowbase + k * 16 + iota
            pltpu.make_async_copy(img_ref.at[idxbuf[s]], bufP[s],
                                  semP[s]).start()

    def wait_in(t, s):
        kind, p, g0 = tinfo(t)

        @pl.when(kind < 2)
        def _a():
            pltpu.make_async_copy(img_ref.at[pl.ds(g0, 48)],
                                  bufIn[s].at[pl.ds(0, 48)], semIn[s]).wait()

        @pl.when(kind >= 2)
        def _b():
            pltpu.make_async_copy(img_ref.at[pl.ds(g0, 64)],
                                  bufIn[s].at[pl.ds(0, 64)], semIn[s]).wait()
            pltpu.make_async_copy(img_ref.at[idxbuf[s]], bufP[s],
                                  semP[s]).wait()

    def blend(t, s):
        kind, _, _ = tinfo(t)

        @pl.when(kind >= 2)
        def _b():
            j0 = jnp.where(kind == 2, _Y0 - _RS, 0)

            def brow(j, carry):
                for k in range(_X1 // 16):
                    bufIn[s][j, pl.ds(k * 16, 16)] = bufP[s][j, pl.ds(k * 16, 16)]
                ktail = (_X1 // 16) * 16
                vp = bufP[s][j, pl.ds(ktail, 16)]
                vs = bufIn[s][j, pl.ds(ktail, 16)]
                bufIn[s][j, pl.ds(ktail, 16)] = jnp.where(
                    iota < _X1 - ktail, vp, vs)
                return carry
            lax.fori_loop(j0, 64, brow, 0)

    def make_out(t, s):
        kind, p, g0 = tinfo(t)
        return kind, pltpu.make_async_copy(
            bufIn[s].at[pl.ds(0, 48)], out_ref.at[pl.ds(g0, 48)], semOut[s]), \
            pltpu.make_async_copy(
            bufIn[s].at[pl.ds(0, 64)], out_ref.at[pl.ds(g0, 64)], semOut[s])

    def start_out(t, s):
        kind, cp48, cp64 = make_out(t, s)
        pl.when(kind < 2)(lambda: cp48.start())
        pl.when(kind >= 2)(lambda: cp64.start())

    def wait_out(t, s):
        kind, cp48, cp64 = make_out(t, s)
        pl.when(kind < 2)(lambda: cp48.wait())
        pl.when(kind >= 2)(lambda: cp64.wait())

    T = TPW
    start_in(0, 0)
    start_in(1, 1)

    def iter_g(g, carry):
        for s in range(_NS):
            t = g * _NS + s
            wait_in(t, s)
            blend(t, s)
            start_out(t, s)
            s2 = (s + 2) % _NS
            t2 = t + 2

            @pl.when(t2 < T)
            def _pf():
                @pl.when(t2 >= _NS)
                def _w():
                    wait_out(t - 2, s2)
                start_in(t2, s2)
        return carry
    lax.fori_loop(0, T // _NS, iter_g, 0)

    for s in range(_NS):
        wait_out(T - _NS + s, s)

    @pl.when(wid == 0)
    def _labels():
        pltpu.sync_copy(labels_ref, labelsv)
        for k in range(B // 16):
            idxv = indexv[pl.ds(k * 16, 16)]
            laboutv[pl.ds(k * 16, 16)] = plsc.load_gather(labelsv, [idxv])
        pltpu.sync_copy(laboutv, lab_out_ref)


def kernel(images, labels, index):
    B, C, H, W = images.shape
    x0, y0, x1, y1 = _cut_box(H, W, alpha=1.0, seed=0)
    assert (x0, y0, x1, y1) == (0, _Y0, _X1, H)

    info = plsc.get_sparse_core_info()
    NW = info.num_cores * info.num_subcores
    TPW = (B * C // NW) * 4

    img2d = images.reshape(B * C * H, W)
    mesh = plsc.VectorSubcoreMesh(core_axis_name="c", subcore_axis_name="s")

    scratch = (
        [pltpu.VMEM((64, W), jnp.float32) for _ in range(_NS)] +
        [pltpu.VMEM((64, W), jnp.float32) for _ in range(_NS)] +
        [pltpu.VMEM((64,), jnp.int32) for _ in range(_NS)] +
        [pltpu.VMEM((B,), jnp.int32) for _ in range(3)] +
        [pltpu.SemaphoreType.DMA for _ in range(3 * _NS)]
    )

    sc = pl.kernel(
        functools.partial(_sc_body, B, C, H, W, TPW),
        out_type=[
            jax.ShapeDtypeStruct((B * C * H, W), images.dtype),
            jax.ShapeDtypeStruct((B,), labels.dtype),
        ],
        mesh=mesh,
        scratch_types=scratch,
        compiler_params=pltpu.CompilerParams(
            needs_layout_passes=False, use_tc_tiling_on_sc=False),
    )
    out2d, labels_b = sc(img2d, index, labels)
    mixed = out2d.reshape(B, C, H, W)

    lam = 1.0 - (x1 - x0) * (y1 - y0) / (W * H)
    return (mixed, labels, labels_b, jnp.float32(lam))


# speedup vs baseline: 2.0514x; 1.9110x over previous
import functools

import jax
import jax.numpy as jnp
import numpy as np
from jax import lax
from jax.experimental import pallas as pl
from jax.experimental.pallas import tpu as pltpu
from jax.experimental.pallas import tpu_sc as plsc


def _cut_box(H, W, alpha=1.0, seed=0):
    rng = np.random.RandomState(seed)
    lam = rng.beta(alpha, alpha)
    cx = rng.uniform(0, W)
    cy = rng.uniform(0, H)
    w = W * np.sqrt(1.0 - lam)
    h = H * np.sqrt(1.0 - lam)
    x0 = int(np.clip(cx - w // 2, 0, W))
    y0 = int(np.clip(cy - h // 2, 0, H))
    x1 = int(np.clip(cx + w // 2, 0, W))
    y1 = int(np.clip(cy + h // 2, 0, H))
    return x0, y0, x1, y1


_Y0 = 103
_X1 = 87
_NS = 4
_BPP = 28


def _sc_body(B, C, H, W, TPW, img_ref, index_ref, labels_ref,
             out_ref, lab_out_ref, *scratch):
    bufIn = scratch[0:_NS]
    bufP = scratch[_NS:2 * _NS]
    indexv, labelsv, laboutv = scratch[2 * _NS:2 * _NS + 3]
    semIn = scratch[2 * _NS + 3:3 * _NS + 3]
    semP = scratch[3 * _NS + 3:4 * _NS + 3]
    semOut = scratch[4 * _NS + 3:5 * _NS + 3]

    nc = plsc.get_sparse_core_info().num_cores
    wid = lax.axis_index("s") * nc + lax.axis_index("c")
    planes_per_w = TPW // 4
    iota = lax.iota(jnp.int32, 16)

    pltpu.sync_copy(index_ref, indexv)

    def tinfo(t):
        kind = t % 4
        p = wid * planes_per_w + t // 4
        bandoff = jnp.where(kind < 2, 6 * kind, 8 * kind - 4)
        return kind, p, p * _BPP + bandoff

    def perm_src(t):
        kind, p, _ = tinfo(t)
        b = p // C
        c = p - b * C
        srcb = jnp.max(plsc.load_gather(
            indexv, [jnp.full((16,), b, jnp.int32)]))
        return (srcb * C + c) * _BPP + (8 * kind - 4)

    def start_in(t, s):
        kind, p, g0 = tinfo(t)

        @pl.when(kind < 2)
        def _a():
            pltpu.make_async_copy(img_ref.at[pl.ds(g0, 6)],
                                  bufIn[s].at[pl.ds(0, 6)], semIn[s]).start()

        @pl.when(kind >= 2)
        def _b():
            pltpu.make_async_copy(img_ref.at[pl.ds(g0, 8)],
                                  bufIn[s], semIn[s]).start()
            sb = perm_src(t)
            pltpu.make_async_copy(
                img_ref.at[pl.ds(sb, 8), :, pl.ds(0, 128)],
                bufP[s], semP[s]).start()

    def wait_in(t, s):
        kind, p, g0 = tinfo(t)

        @pl.when(kind < 2)
        def _a():
            pltpu.make_async_copy(img_ref.at[pl.ds(g0, 6)],
                                  bufIn[s].at[pl.ds(0, 6)], semIn[s]).wait()

        @pl.when(kind >= 2)
        def _b():
            pltpu.make_async_copy(img_ref.at[pl.ds(g0, 8)],
                                  bufIn[s], semIn[s]).wait()
            sb = perm_src(t)
            pltpu.make_async_copy(
                img_ref.at[pl.ds(sb, 8), :, pl.ds(0, 128)],
                bufP[s], semP[s]).wait()

    def blend(t, s):
        kind, _, _ = tinfo(t)

        @pl.when(kind >= 2)
        def _b():
            j0 = jnp.where(kind == 2, _Y0 - 96, 0)

            def brow(j, carry):
                jb = j // 8
                r = j - jb * 8
                for k in range(_X1 // 16):
                    bufIn[s][jb, r, pl.ds(k * 16, 16)] = (
                        bufP[s][jb, r, pl.ds(k * 16, 16)])
                ktail = (_X1 // 16) * 16
                vp = bufP[s][jb, r, pl.ds(ktail, 16)]
                vs = bufIn[s][jb, r, pl.ds(ktail, 16)]
                bufIn[s][jb, r, pl.ds(ktail, 16)] = jnp.where(
                    iota < _X1 - ktail, vp, vs)
                return carry
            lax.fori_loop(j0, 64, brow, 0)

    def make_out(t, s):
        kind, p, g0 = tinfo(t)
        return kind, pltpu.make_async_copy(
            bufIn[s].at[pl.ds(0, 6)], out_ref.at[pl.ds(g0, 6)], semOut[s]), \
            pltpu.make_async_copy(bufIn[s], out_ref.at[pl.ds(g0, 8)],
                                  semOut[s])

    def start_out(t, s):
        kind, cp6, cp8 = make_out(t, s)
        pl.when(kind < 2)(lambda: cp6.start())
        pl.when(kind >= 2)(lambda: cp8.start())

    def wait_out(t, s):
        kind, cp6, cp8 = make_out(t, s)
        pl.when(kind < 2)(lambda: cp6.wait())
        pl.when(kind >= 2)(lambda: cp8.wait())

    T = TPW
    start_in(0, 0)
    start_in(1, 1)

    def iter_g(g, carry):
        for s in range(_NS):
            t = g * _NS + s
            wait_in(t, s)
            blend(t, s)
            start_out(t, s)
            s2 = (s + 2) % _NS
            t2 = t + 2

            @pl.when(t2 < T)
            def _pf():
                @pl.when(t2 >= _NS)
                def _w():
                    wait_out(t2 - _NS, s2)
                start_in(t2, s2)
        return carry
    lax.fori_loop(0, T // _NS, iter_g, 0)

    for s in range(_NS):
        wait_out(T - _NS + s, (T - _NS + s) % _NS)

    @pl.when(wid == 0)
    def _labels():
        pltpu.sync_copy(labels_ref, labelsv)
        for k in range(B // 16):
            idxv = indexv[pl.ds(k * 16, 16)]
            laboutv[pl.ds(k * 16, 16)] = plsc.load_gather(labelsv, [idxv])
        pltpu.sync_copy(laboutv, lab_out_ref)


def kernel(images, labels, index):
    B, C, H, W = images.shape
    x0, y0, x1, y1 = _cut_box(H, W, alpha=1.0, seed=0)
    assert (x0, y0, x1, y1) == (0, _Y0, _X1, H)

    info = plsc.get_sparse_core_info()
    NW = info.num_cores * info.num_subcores
    TPW = (B * C // NW) * 4

    img3 = images.reshape(B * C * _BPP, 8, W)
    mesh = plsc.VectorSubcoreMesh(core_axis_name="c", subcore_axis_name="s")

    scratch = (
        [pltpu.VMEM((8, 8, W), jnp.float32) for _ in range(_NS)] +
        [pltpu.VMEM((8, 8, 128), jnp.float32) for _ in range(_NS)] +
        [pltpu.VMEM((B,), jnp.int32) for _ in range(3)] +
        [pltpu.SemaphoreType.DMA for _ in range(3 * _NS)]
    )

    sc = pl.kernel(
        functools.partial(_sc_body, B, C, H, W, TPW),
        out_type=[
            jax.ShapeDtypeStruct((B * C * _BPP, 8, W), images.dtype),
            jax.ShapeDtypeStruct((B,), labels.dtype),
        ],
        mesh=mesh,
        scratch_types=scratch,
        compiler_params=pltpu.CompilerParams(
            needs_layout_passes=False, use_tc_tiling_on_sc=True),
    )
    out3, labels_b = sc(img3, index, labels)
    mixed = out3.reshape(B, C, H, W)

    lam = 1.0 - (x1 - x0) * (y1 - y0) / (W * H)
    return (mixed, labels, labels_b, jnp.float32(lam))
